# baseline (speedup 1.0000x reference)
import functools

import jax
import jax.numpy as jnp
import numpy as np
from jax import lax
from jax.experimental import pallas as pl
from jax.experimental.pallas import tpu as pltpu
from jax.experimental.pallas import tpu_sc as plsc

B = 32
SEQ = 96
ENC = 120
E = 1920
LBL = 48
D = 1920
NHEAD = 3
DH = D // NHEAD
NLAYERS = 4
DFF = 512
NP = 128
FP = 128
HC = 960
NST = 11


_SC_MESH = plsc.VectorSubcoreMesh(core_axis_name="c", subcore_axis_name="s",
                                  num_cores=2, num_subcores=16)


@functools.partial(
    pl.kernel,
    out_type=jax.ShapeDtypeStruct((NP, NP), jnp.float32),
    mesh=_SC_MESH,
    scratch_types=[
        pltpu.VMEM((E,), jnp.int32),
        pltpu.VMEM((E,), jnp.int32),
        pltpu.VMEM((E,), jnp.float32),
        pltpu.VMEM((NP, NP), jnp.float32),
        pltpu.VMEM((16 * NP,), jnp.float32),
        pltpu.VMEM((NP,), jnp.float32),
    ],
    compiler_params=pltpu.CompilerParams(needs_layout_passes=False),
)
def _a_build_sc(ei_hbm, ew_hbm, out_hbm, src_v, dst_v, ew_v, a2d,
                degp, dinv_v):
    cid = lax.axis_index("c")
    sid = lax.axis_index("s")

    @pl.when((cid == 0) & (sid == 0))
    def _work():
        pltpu.sync_copy(ei_hbm.at[0], src_v)
        pltpu.sync_copy(ei_hbm.at[1], dst_v)
        pltpu.sync_copy(ew_hbm, ew_v)

        lane = lax.broadcasted_iota(jnp.int32, (16,), 0)
        zeros16 = jnp.zeros((16,), jnp.float32)

        @plsc.parallel_loop(0, NP * NP // 16, unroll=8)
        def _zero_a(i):
            a2d[i // 8, pl.ds((i % 8) * 16, 16)] = zeros16

        @plsc.parallel_loop(0, 16 * NP // 16, unroll=8)
        def _zero_d(i):
            degp[pl.ds(i * 16, 16)] = zeros16

        lane_off = lane * NP

        @plsc.parallel_loop(0, E // 16, unroll=4)
        def _deg(cidx):
            d16 = dst_v[pl.ds(cidx * 16, 16)]
            w16 = ew_v[pl.ds(cidx * 16, 16)]
            plsc.addupdate_scatter(degp, [d16 + lane_off], w16)

        for cc in range(NP // 16):
            acc = zeros16
            for r in range(16):
                acc = acc + degp[pl.ds(r * NP + cc * 16, 16)]
            d = acc + 1.0
            y = plsc.bitcast(
                jnp.int32(0x5F3759DF) - (plsc.bitcast(d, jnp.int32) >> 1),
                jnp.float32)
            for _ in range(3):
                y = y * (1.5 - 0.5 * d * y * y)
            dinv_v[pl.ds(cc * 16, 16)] = jnp.where(d > 0.0, y, 0.0)

        @plsc.parallel_loop(0, E // 16, unroll=2)
        def _edges(cidx):
            s16 = src_v[pl.ds(cidx * 16, 16)]
            d16 = dst_v[pl.ds(cidx * 16, 16)]
            w16 = ew_v[pl.ds(cidx * 16, 16)]
            ds_g = plsc.load_gather(dinv_v, [d16])
            ss_g = plsc.load_gather(dinv_v, [s16])
            n16 = ss_g * w16 * ds_g
            for j in range(16):
                plsc.addupdate_scatter(a2d, [d16, s16], n16, mask=(lane == j))

        for cc in range(NP // 16):
            n16 = lane + cc * 16
            dv = dinv_v[pl.ds(cc * 16, 16)]
            plsc.addupdate_scatter(a2d, [n16, n16], dv * dv)

        pltpu.sync_copy(a2d, out_hbm)


def _gcn_body(x_ref, a_ref, w1_ref, b1_ref, w2_ref, b2_ref,
              w3_ref, b3_ref, w4_ref, b4_ref, out_ref):
    a_mat = a_ref[...]
    x3 = x_ref[...]
    x2 = x3.reshape(NP * B, FP)
    for w_ref, b_ref in ((w1_ref, b1_ref), (w2_ref, b2_ref),
                         (w3_ref, b3_ref), (w4_ref, b4_ref)):
        wr = w_ref[...]
        o, ci = wr.shape
        w = jnp.concatenate([
            jnp.concatenate([wr, jnp.zeros((o, FP - ci), jnp.float32)], 1),
            jnp.zeros((FP - o, FP), jnp.float32)], 0)
        bvec = jnp.concatenate(
            [b_ref[...], jnp.zeros((1, FP - o), jnp.float32)], 1)
        y = lax.dot_general(x2, w, (((1,), (1,)), ((), ())),
                            preferred_element_type=jnp.float32)
        yv = y.reshape(NP, B * FP)
        z = lax.dot_general(a_mat, yv, (((1,), (0,)), ((), ())),
                            preferred_element_type=jnp.float32)
        x2 = jnp.maximum(z.reshape(NP * B, FP) + bvec, 0.0)
    out_ref[...] = x2.reshape(NP, B, FP)

def _graph_gcn_call(x3, a_mat, w1p, b1p, w2p, b2p, w3p, b3p, w4p, b4p):
    return pl.pallas_call(
        _gcn_body,
        out_shape=jax.ShapeDtypeStruct((NP, B, FP), jnp.float32),
    )(x3, a_mat, w1p, b1p, w2p, b2p, w3p, b3p, w4p, b4p)


def _ln(x, w, b):
    mu = jnp.mean(x, axis=-1, keepdims=True)
    var = jnp.mean((x - mu) ** 2, axis=-1, keepdims=True)
    return (x - mu) / jnp.sqrt(var + 1e-5) * w + b


def _tr_body(h_ref, win_ref, wout_ref, w1_ref, w2_ref, inb_ref, outb_ref,
             l1b_ref, l2b_ref, ln1w_ref, ln1b_ref, ln2w_ref, ln2b_ref,
             ow_ref, ob_ref, out_ref, h_s, qkv_s, o_s, pj_s, ff_s):
    g = pl.program_id(0)
    s = g % NST
    l = g // NST

    @pl.when(g == 0)
    def _init():
        h_s[...] = h_ref[...]

    @pl.when(s < 6)
    def _qkv():
        part = lax.dot_general(h_s[...], win_ref[0],
                               (((1,), (1,)), ((), ())),
                               preferred_element_type=jnp.float32)
        qkv_s[pl.ds(s, 1)] = part[None]

    @pl.when(s == 6)
    def _attn():
        scale = jnp.float32(np.sqrt(DH).astype(np.float32))
        lrow = pl.ds(l, 1)
        q = jnp.concatenate([qkv_s[0], qkv_s[1]], axis=1) + inb_ref[lrow, :D]
        k = (jnp.concatenate([qkv_s[2], qkv_s[3]], axis=1)
             + inb_ref[lrow, D:2 * D])
        v = (jnp.concatenate([qkv_s[4], qkv_s[5]], axis=1)
             + inb_ref[lrow, 2 * D:3 * D])
        outs = []
        for hh in range(NHEAD):
            qh = q[:, hh * DH:(hh + 1) * DH]
            kh = k[:, hh * DH:(hh + 1) * DH]
            vh = v[:, hh * DH:(hh + 1) * DH]
            logits = lax.dot_general(qh, kh, (((1,), (1,)), ((), ())),
                                     preferred_element_type=jnp.float32)
            logits = logits / scale
            m = jnp.max(logits, axis=-1, keepdims=True)
            e = jnp.exp(logits - m)
            att = e / jnp.sum(e, axis=-1, keepdims=True)
            outs.append(lax.dot_general(att, vh, (((1,), (0,)), ((), ())),
                                        preferred_element_type=jnp.float32))
        o_s[...] = jnp.concatenate(outs, axis=1)

    @pl.when((s == 7) | (s == 8))
    def _proj():
        pj = lax.dot_general(o_s[...], wout_ref[0], (((1,), (1,)), ((), ())),
                             preferred_element_type=jnp.float32)
        pj_s[pl.ds(s - 7, 1)] = pj[None]

        @pl.when(s == 8)
        def _res1():
            lrow = pl.ds(l, 1)
            pj_full = (jnp.concatenate([pj_s[0], pj_s[1]], axis=1)
                       + outb_ref[lrow, :])
            h_s[...] = _ln(h_s[...] + pj_full, ln1w_ref[lrow, :],
                           ln1b_ref[lrow, :])

    @pl.when(s == 9)
    def _ff1():
        hid = lax.dot_general(h_s[...], w1_ref[0], (((1,), (1,)), ((), ())),
                              preferred_element_type=jnp.float32)
        ff_s[...] = jnp.maximum(hid + l1b_ref[pl.ds(l, 1), :], 0.0)

    @pl.when(s == 10)
    def _ff2():
        lrow = pl.ds(l, 1)
        ff2 = lax.dot_general(ff_s[...], w2_ref[0], (((1,), (1,)), ((), ())),
                              preferred_element_type=jnp.float32)
        ff2 = ff2 + l2b_ref[lrow, :]
        h2 = _ln(h_s[...] + ff2, ln2w_ref[lrow, :], ln2b_ref[lrow, :])
        h_s[...] = h2

        @pl.when(g == NST * NLAYERS - 1)
        def _final():
            res = lax.dot_general(h2, ow_ref[...], (((1,), (1,)), ((), ())),
                                  preferred_element_type=jnp.float32)
            out_ref[...] = res + ob_ref[...]


def _transformer_call(h, win_r, wout_r, w1, w2, inb, outb, l1b, l2b,
                      ln1w, ln1b, ln2w, ln2b, ow, ob2):
    nsteps = NST * NLAYERS
    const2 = lambda g: (0, 0)
    win_idx = lambda g: (6 * (g // NST) + jnp.minimum(g % NST, 5), 0, 0)
    wout_idx = lambda g: (jnp.where(
        g % NST >= 8, 2 * (g // NST) + 1,
        jnp.where(g % NST >= 7, 2 * (g // NST),
                  jnp.maximum(2 * (g // NST) - 1, 0))), 0, 0)
    w1_idx = lambda g: (jnp.where(g % NST >= 9, g // NST,
                                  jnp.maximum(g // NST - 1, 0)), 0, 0)
    w2_idx = lambda g: (jnp.where(g % NST >= 10, g // NST,
                                  jnp.maximum(g // NST - 1, 0)), 0, 0)
    in_specs = [
        pl.BlockSpec((B, D), const2),
        pl.BlockSpec((1, HC, D), win_idx),
        pl.BlockSpec((1, HC, D), wout_idx),
        pl.BlockSpec((1, DFF, D), w1_idx),
        pl.BlockSpec((1, D, DFF), w2_idx),
        pl.BlockSpec((NLAYERS, 3 * D), const2),
        pl.BlockSpec((NLAYERS, D), const2),
        pl.BlockSpec((NLAYERS, DFF), const2),
        pl.BlockSpec((NLAYERS, D), const2),
        pl.BlockSpec((NLAYERS, D), const2),
        pl.BlockSpec((NLAYERS, D), const2),
        pl.BlockSpec((NLAYERS, D), const2),
        pl.BlockSpec((NLAYERS, D), const2),
        pl.BlockSpec((LBL, D), const2),
        pl.BlockSpec((1, LBL), const2),
    ]
    return pl.pallas_call(
        _tr_body,
        grid=(nsteps,),
        in_specs=in_specs,
        out_specs=pl.BlockSpec((B, LBL), const2),
        out_shape=jax.ShapeDtypeStruct((B, LBL), jnp.float32),
        scratch_shapes=[
            pltpu.VMEM((B, D), jnp.float32),
            pltpu.VMEM((6, B, HC), jnp.float32),
            pltpu.VMEM((B, D), jnp.float32),
            pltpu.VMEM((2, B, HC), jnp.float32),
            pltpu.VMEM((B, DFF), jnp.float32),
        ],
        compiler_params=pltpu.CompilerParams(
            dimension_semantics=("arbitrary",),
        ),
    )(h, win_r, wout_r, w1, w2, inb, outb, l1b, l2b, ln1w, ln1b, ln2w, ln2b,
      ow, ob2)


def kernel(inputs, edge_index, edge_attr, gcn_w1, gcn_b1, gcn_w2, gcn_b2,
           gcn_w3, gcn_b3, gcn_w4, gcn_b4, tr_in_w, tr_in_b, tr_out_w,
           tr_out_b, tr_l1_w, tr_l1_b, tr_l2_w, tr_l2_b, tr_ln1_w, tr_ln1_b,
           tr_ln2_w, tr_ln2_b, out_w, out_b):
    x3 = jnp.transpose(inputs, (2, 0, 1))
    x3 = jnp.pad(x3, ((0, NP - ENC), (0, 0), (0, FP - SEQ)))
    a_mat = _a_build_sc(edge_index, edge_attr)

    z4 = _graph_gcn_call(x3, a_mat,
                         gcn_w1, gcn_b1.reshape(1, -1),
                         gcn_w2, gcn_b2.reshape(1, -1),
                         gcn_w3, gcn_b3.reshape(1, -1),
                         gcn_w4, gcn_b4.reshape(1, -1))
    h = z4[:ENC, :, :SEQ // 6].transpose(1, 0, 2).reshape(B, D)

    win_r = tr_in_w.reshape(NLAYERS * 6, HC, D)
    wout_r = tr_out_w.reshape(NLAYERS * 2, HC, D)
    out = _transformer_call(
        h, win_r, wout_r, tr_l1_w, tr_l2_w, tr_in_b, tr_out_b, tr_l1_b,
        tr_l2_b, tr_ln1_w, tr_ln1_b, tr_ln2_w, tr_ln2_b, out_w,
        out_b.reshape(1, LBL))
    return out.reshape(B, 1, LBL)


# speedup vs baseline: 1.0114x; 1.0114x over previous
import functools

import jax
import jax.numpy as jnp
import numpy as np
from jax import lax
from jax.experimental import pallas as pl
from jax.experimental.pallas import tpu as pltpu
from jax.experimental.pallas import tpu_sc as plsc

B = 32
SEQ = 96
ENC = 120
E = 1920
LBL = 48
D = 1920
NHEAD = 3
DH = D // NHEAD
NLAYERS = 4
DFF = 512
NP = 128
FP = 128
HC = 960
NST = 11


_SC_MESH = plsc.VectorSubcoreMesh(core_axis_name="c", subcore_axis_name="s",
                                  num_cores=1, num_subcores=16)


@functools.partial(
    pl.kernel,
    out_type=jax.ShapeDtypeStruct((NP, NP), jnp.float32),
    mesh=_SC_MESH,
    scratch_types=[
        pltpu.VMEM((E,), jnp.int32),
        pltpu.VMEM((E,), jnp.int32),
        pltpu.VMEM((E,), jnp.float32),
        pltpu.VMEM((NP, NP), jnp.float32),
        pltpu.VMEM((16 * NP,), jnp.float32),
        pltpu.VMEM((NP,), jnp.float32),
    ],
    compiler_params=pltpu.CompilerParams(needs_layout_passes=False),
)
def _a_build_sc(ei_hbm, ew_hbm, out_hbm, src_v, dst_v, ew_v, a2d,
                degp, dinv_v):
    cid = lax.axis_index("c")
    sid = lax.axis_index("s")

    @pl.when((cid == 0) & (sid == 0))
    def _work():
        pltpu.sync_copy(ei_hbm.at[0], src_v)
        pltpu.sync_copy(ei_hbm.at[1], dst_v)
        pltpu.sync_copy(ew_hbm, ew_v)

        lane = lax.broadcasted_iota(jnp.int32, (16,), 0)
        zeros16 = jnp.zeros((16,), jnp.float32)

        @plsc.parallel_loop(0, NP * NP // 16, unroll=8)
        def _zero_a(i):
            a2d[i // 8, pl.ds((i % 8) * 16, 16)] = zeros16

        @plsc.parallel_loop(0, 16 * NP // 16, unroll=8)
        def _zero_d(i):
            degp[pl.ds(i * 16, 16)] = zeros16

        lane_off = lane * NP

        @plsc.parallel_loop(0, E // 16, unroll=4)
        def _deg(cidx):
            d16 = dst_v[pl.ds(cidx * 16, 16)]
            w16 = ew_v[pl.ds(cidx * 16, 16)]
            plsc.addupdate_scatter(degp, [d16 + lane_off], w16)

        for cc in range(NP // 16):
            acc = zeros16
            for r in range(16):
                acc = acc + degp[pl.ds(r * NP + cc * 16, 16)]
            d = acc + 1.0
            y = plsc.bitcast(
                jnp.int32(0x5F3759DF) - (plsc.bitcast(d, jnp.int32) >> 1),
                jnp.float32)
            for _ in range(3):
                y = y * (1.5 - 0.5 * d * y * y)
            dinv_v[pl.ds(cc * 16, 16)] = jnp.where(d > 0.0, y, 0.0)

        @plsc.parallel_loop(0, E // 16, unroll=2)
        def _edges(cidx):
            s16 = src_v[pl.ds(cidx * 16, 16)]
            d16 = dst_v[pl.ds(cidx * 16, 16)]
            w16 = ew_v[pl.ds(cidx * 16, 16)]
            ds_g = plsc.load_gather(dinv_v, [d16])
            ss_g = plsc.load_gather(dinv_v, [s16])
            n16 = ss_g * w16 * ds_g
            for j in range(16):
                plsc.addupdate_scatter(a2d, [d16, s16], n16, mask=(lane == j))

        for cc in range(NP // 16):
            n16 = lane + cc * 16
            dv = dinv_v[pl.ds(cc * 16, 16)]
            plsc.addupdate_scatter(a2d, [n16, n16], dv * dv)

        pltpu.sync_copy(a2d, out_hbm)


def _gcn_body(x_ref, a_ref, w1_ref, b1_ref, w2_ref, b2_ref,
              w3_ref, b3_ref, w4_ref, b4_ref, out_ref):
    a_mat = a_ref[...]
    x3 = x_ref[...]
    x2 = x3.reshape(NP * B, FP)
    for w_ref, b_ref in ((w1_ref, b1_ref), (w2_ref, b2_ref),
                         (w3_ref, b3_ref), (w4_ref, b4_ref)):
        wr = w_ref[...]
        o, ci = wr.shape
        w = jnp.concatenate([
            jnp.concatenate([wr, jnp.zeros((o, FP - ci), jnp.float32)], 1),
            jnp.zeros((FP - o, FP), jnp.float32)], 0)
        bvec = jnp.concatenate(
            [b_ref[...], jnp.zeros((1, FP - o), jnp.float32)], 1)
        y = lax.dot_general(x2, w, (((1,), (1,)), ((), ())),
                            preferred_element_type=jnp.float32)
        yv = y.reshape(NP, B * FP)
        z = lax.dot_general(a_mat, yv, (((1,), (0,)), ((), ())),
                            preferred_element_type=jnp.float32)
        x2 = jnp.maximum(z.reshape(NP * B, FP) + bvec, 0.0)
    out_ref[...] = x2.reshape(NP, B, FP)

def _graph_gcn_call(x3, a_mat, w1p, b1p, w2p, b2p, w3p, b3p, w4p, b4p):
    return pl.pallas_call(
        _gcn_body,
        out_shape=jax.ShapeDtypeStruct((NP, B, FP), jnp.float32),
    )(x3, a_mat, w1p, b1p, w2p, b2p, w3p, b3p, w4p, b4p)


def _ln(x, w, b):
    mu = jnp.mean(x, axis=-1, keepdims=True)
    var = jnp.mean((x - mu) ** 2, axis=-1, keepdims=True)
    return (x - mu) / jnp.sqrt(var + 1e-5) * w + b


def _tr_body(h_ref, win_ref, wout_ref, w1_ref, w2_ref, inb_ref, outb_ref,
             l1b_ref, l2b_ref, ln1w_ref, ln1b_ref, ln2w_ref, ln2b_ref,
             ow_ref, ob_ref, out_ref, h_s, qkv_s, o_s, pj_s, ff_s):
    g = pl.program_id(0)
    s = g % NST
    l = g // NST

    @pl.when(g == 0)
    def _init():
        h_s[...] = h_ref[...]

    @pl.when(s < 6)
    def _qkv():
        part = lax.dot_general(h_s[...], win_ref[0],
                               (((1,), (1,)), ((), ())),
                               preferred_element_type=jnp.float32)
        qkv_s[pl.ds(s, 1)] = part[None]

    @pl.when(s == 6)
    def _attn():
        scale = jnp.float32(np.sqrt(DH).astype(np.float32))
        lrow = pl.ds(l, 1)
        q = jnp.concatenate([qkv_s[0], qkv_s[1]], axis=1) + inb_ref[lrow, :D]
        k = (jnp.concatenate([qkv_s[2], qkv_s[3]], axis=1)
             + inb_ref[lrow, D:2 * D])
        v = (jnp.concatenate([qkv_s[4], qkv_s[5]], axis=1)
             + inb_ref[lrow, 2 * D:3 * D])
        outs = []
        for hh in range(NHEAD):
            qh = q[:, hh * DH:(hh + 1) * DH]
            kh = k[:, hh * DH:(hh + 1) * DH]
            vh = v[:, hh * DH:(hh + 1) * DH]
            logits = lax.dot_general(qh, kh, (((1,), (1,)), ((), ())),
                                     preferred_element_type=jnp.float32)
            logits = logits / scale
            m = jnp.max(logits, axis=-1, keepdims=True)
            e = jnp.exp(logits - m)
            att = e / jnp.sum(e, axis=-1, keepdims=True)
            outs.append(lax.dot_general(att, vh, (((1,), (0,)), ((), ())),
                                        preferred_element_type=jnp.float32))
        o_s[...] = jnp.concatenate(outs, axis=1)

    @pl.when((s == 7) | (s == 8))
    def _proj():
        pj = lax.dot_general(o_s[...], wout_ref[0], (((1,), (1,)), ((), ())),
                             preferred_element_type=jnp.float32)
        pj_s[pl.ds(s - 7, 1)] = pj[None]

        @pl.when(s == 8)
        def _res1():
            lrow = pl.ds(l, 1)
            pj_full = (jnp.concatenate([pj_s[0], pj_s[1]], axis=1)
                       + outb_ref[lrow, :])
            h_s[...] = _ln(h_s[...] + pj_full, ln1w_ref[lrow, :],
                           ln1b_ref[lrow, :])

    @pl.when(s == 9)
    def _ff1():
        hid = lax.dot_general(h_s[...], w1_ref[0], (((1,), (1,)), ((), ())),
                              preferred_element_type=jnp.float32)
        ff_s[...] = jnp.maximum(hid + l1b_ref[pl.ds(l, 1), :], 0.0)

    @pl.when(s == 10)
    def _ff2():
        lrow = pl.ds(l, 1)
        ff2 = lax.dot_general(ff_s[...], w2_ref[0], (((1,), (1,)), ((), ())),
                              preferred_element_type=jnp.float32)
        ff2 = ff2 + l2b_ref[lrow, :]
        h2 = _ln(h_s[...] + ff2, ln2w_ref[lrow, :], ln2b_ref[lrow, :])
        h_s[...] = h2

        @pl.when(g == NST * NLAYERS - 1)
        def _final():
            res = lax.dot_general(h2, ow_ref[...], (((1,), (1,)), ((), ())),
                                  preferred_element_type=jnp.float32)
            out_ref[...] = res + ob_ref[...]


def _transformer_call(h, win_r, wout_r, w1, w2, inb, outb, l1b, l2b,
                      ln1w, ln1b, ln2w, ln2b, ow, ob2):
    nsteps = NST * NLAYERS
    const2 = lambda g: (0, 0)
    win_idx = lambda g: (6 * (g // NST) + jnp.minimum(g % NST, 5), 0, 0)
    wout_idx = lambda g: (jnp.where(
        g % NST >= 8, 2 * (g // NST) + 1,
        jnp.where(g % NST >= 7, 2 * (g // NST),
                  jnp.maximum(2 * (g // NST) - 1, 0))), 0, 0)
    w1_idx = lambda g: (jnp.where(g % NST >= 9, g // NST,
                                  jnp.maximum(g // NST - 1, 0)), 0, 0)
    w2_idx = lambda g: (jnp.where(g % NST >= 10, g // NST,
                                  jnp.maximum(g // NST - 1, 0)), 0, 0)
    in_specs = [
        pl.BlockSpec((B, D), const2),
        pl.BlockSpec((1, HC, D), win_idx),
        pl.BlockSpec((1, HC, D), wout_idx),
        pl.BlockSpec((1, DFF, D), w1_idx),
        pl.BlockSpec((1, D, DFF), w2_idx),
        pl.BlockSpec((NLAYERS, 3 * D), const2),
        pl.BlockSpec((NLAYERS, D), const2),
        pl.BlockSpec((NLAYERS, DFF), const2),
        pl.BlockSpec((NLAYERS, D), const2),
        pl.BlockSpec((NLAYERS, D), const2),
        pl.BlockSpec((NLAYERS, D), const2),
        pl.BlockSpec((NLAYERS, D), const2),
        pl.BlockSpec((NLAYERS, D), const2),
        pl.BlockSpec((LBL, D), const2),
        pl.BlockSpec((1, LBL), const2),
    ]
    return pl.pallas_call(
        _tr_body,
        grid=(nsteps,),
        in_specs=in_specs,
        out_specs=pl.BlockSpec((B, LBL), const2),
        out_shape=jax.ShapeDtypeStruct((B, LBL), jnp.float32),
        scratch_shapes=[
            pltpu.VMEM((B, D), jnp.float32),
            pltpu.VMEM((6, B, HC), jnp.float32),
            pltpu.VMEM((B, D), jnp.float32),
            pltpu.VMEM((2, B, HC), jnp.float32),
            pltpu.VMEM((B, DFF), jnp.float32),
        ],
        compiler_params=pltpu.CompilerParams(
            dimension_semantics=("arbitrary",),
        ),
    )(h, win_r, wout_r, w1, w2, inb, outb, l1b, l2b, ln1w, ln1b, ln2w, ln2b,
      ow, ob2)


def kernel(inputs, edge_index, edge_attr, gcn_w1, gcn_b1, gcn_w2, gcn_b2,
           gcn_w3, gcn_b3, gcn_w4, gcn_b4, tr_in_w, tr_in_b, tr_out_w,
           tr_out_b, tr_l1_w, tr_l1_b, tr_l2_w, tr_l2_b, tr_ln1_w, tr_ln1_b,
           tr_ln2_w, tr_ln2_b, out_w, out_b):
    x3 = jnp.transpose(inputs, (2, 0, 1))
    x3 = jnp.pad(x3, ((0, NP - ENC), (0, 0), (0, FP - SEQ)))
    a_mat = _a_build_sc(edge_index, edge_attr)

    z4 = _graph_gcn_call(x3, a_mat,
                         gcn_w1, gcn_b1.reshape(1, -1),
                         gcn_w2, gcn_b2.reshape(1, -1),
                         gcn_w3, gcn_b3.reshape(1, -1),
                         gcn_w4, gcn_b4.reshape(1, -1))
    h = z4[:ENC, :, :SEQ // 6].transpose(1, 0, 2).reshape(B, D)

    win_r = tr_in_w.reshape(NLAYERS * 6, HC, D)
    wout_r = tr_out_w.reshape(NLAYERS * 2, HC, D)
    out = _transformer_call(
        h, win_r, wout_r, tr_l1_w, tr_l2_w, tr_in_b, tr_out_b, tr_l1_b,
        tr_l2_b, tr_ln1_w, tr_ln1_b, tr_ln2_w, tr_ln2_b, out_w,
        out_b.reshape(1, LBL))
    return out.reshape(B, 1, LBL)


# speedup vs baseline: 1.0176x; 1.0062x over previous
import functools

import jax
import jax.numpy as jnp
import numpy as np
from jax import lax
from jax.experimental import pallas as pl
from jax.experimental.pallas import tpu as pltpu
from jax.experimental.pallas import tpu_sc as plsc

B = 32
SEQ = 96
ENC = 120
E = 1920
LBL = 48
D = 1920
NHEAD = 3
DH = D // NHEAD
NLAYERS = 4
DFF = 512
NP = 128
FP = 128
HC = 960
NST = 11


_SC_MESH = plsc.VectorSubcoreMesh(core_axis_name="c", subcore_axis_name="s",
                                  num_cores=1, num_subcores=16)


@functools.partial(
    pl.kernel,
    out_type=jax.ShapeDtypeStruct((8, NP, NP), jnp.float32),
    mesh=_SC_MESH,
    scratch_types=[
        pltpu.VMEM((E,), jnp.int32),
        pltpu.VMEM((E,), jnp.int32),
        pltpu.VMEM((E,), jnp.float32),
        pltpu.VMEM((NP, NP), jnp.float32),
        pltpu.VMEM((16 * NP,), jnp.float32),
        pltpu.VMEM((NP,), jnp.float32),
    ],
    compiler_params=pltpu.CompilerParams(needs_layout_passes=False),
)
def _a_build_sc(ei_hbm, ew_hbm, out_hbm, src_v, dst_v, ew_v, a2d,
                degp, dinv_v):
    sid = lax.axis_index("s")

    @pl.when(sid < 8)
    def _work():
        pltpu.sync_copy(ei_hbm.at[0], src_v)
        pltpu.sync_copy(ei_hbm.at[1], dst_v)
        pltpu.sync_copy(ew_hbm, ew_v)

        lane = lax.broadcasted_iota(jnp.int32, (16,), 0)
        zeros16 = jnp.zeros((16,), jnp.float32)

        @plsc.parallel_loop(0, NP * NP // 16, unroll=8)
        def _zero_a(i):
            a2d[i // 8, pl.ds((i % 8) * 16, 16)] = zeros16

        @plsc.parallel_loop(0, 16 * NP // 16, unroll=8)
        def _zero_d(i):
            degp[pl.ds(i * 16, 16)] = zeros16

        lane_off = lane * NP

        @plsc.parallel_loop(0, E // 16, unroll=4)
        def _deg(cidx):
            d16 = dst_v[pl.ds(cidx * 16, 16)]
            w16 = ew_v[pl.ds(cidx * 16, 16)]
            plsc.addupdate_scatter(degp, [d16 + lane_off], w16)

        for cc in range(NP // 16):
            acc = zeros16
            for r in range(16):
                acc = acc + degp[pl.ds(r * NP + cc * 16, 16)]
            d = acc + 1.0
            y = plsc.bitcast(
                jnp.int32(0x5F3759DF) - (plsc.bitcast(d, jnp.int32) >> 1),
                jnp.float32)
            for _ in range(3):
                y = y * (1.5 - 0.5 * d * y * y)
            dinv_v[pl.ds(cc * 16, 16)] = jnp.where(d > 0.0, y, 0.0)

        nchunk = E // 16 // 8

        @plsc.parallel_loop(0, nchunk, unroll=2)
        def _edges(k):
            cidx = sid * nchunk + k
            s16 = src_v[pl.ds(cidx * 16, 16)]
            d16 = dst_v[pl.ds(cidx * 16, 16)]
            w16 = ew_v[pl.ds(cidx * 16, 16)]
            ds_g = plsc.load_gather(dinv_v, [d16])
            ss_g = plsc.load_gather(dinv_v, [s16])
            n16 = ss_g * w16 * ds_g
            for j in range(16):
                plsc.addupdate_scatter(a2d, [d16, s16], n16, mask=(lane == j))

        @pl.when(sid == 0)
        def _diag():
            for cc in range(NP // 16):
                n16 = lane + cc * 16
                dv = dinv_v[pl.ds(cc * 16, 16)]
                plsc.addupdate_scatter(a2d, [n16, n16], dv * dv)

        pltpu.sync_copy(a2d, out_hbm.at[sid])


def _gcn_body(x_ref, a_ref, w1_ref, b1_ref, w2_ref, b2_ref,
              w3_ref, b3_ref, w4_ref, b4_ref, out_ref):
    ap = a_ref[...]
    a_mat = ap[0]
    for t in range(1, 8):
        a_mat = a_mat + ap[t]
    x3 = x_ref[...]
    x2 = x3.reshape(NP * B, FP)
    for w_ref, b_ref in ((w1_ref, b1_ref), (w2_ref, b2_ref),
                         (w3_ref, b3_ref), (w4_ref, b4_ref)):
        wr = w_ref[...]
        o, ci = wr.shape
        w = jnp.concatenate([
            jnp.concatenate([wr, jnp.zeros((o, FP - ci), jnp.float32)], 1),
            jnp.zeros((FP - o, FP), jnp.float32)], 0)
        bvec = jnp.concatenate(
            [b_ref[...], jnp.zeros((1, FP - o), jnp.float32)], 1)
        y = lax.dot_general(x2, w, (((1,), (1,)), ((), ())),
                            preferred_element_type=jnp.float32)
        yv = y.reshape(NP, B * FP)
        z = lax.dot_general(a_mat, yv, (((1,), (0,)), ((), ())),
                            preferred_element_type=jnp.float32)
        x2 = jnp.maximum(z.reshape(NP * B, FP) + bvec, 0.0)
    out_ref[...] = x2.reshape(NP, B, FP)

def _graph_gcn_call(x3, a_mat, w1p, b1p, w2p, b2p, w3p, b3p, w4p, b4p):
    return pl.pallas_call(
        _gcn_body,
        out_shape=jax.ShapeDtypeStruct((NP, B, FP), jnp.float32),
    )(x3, a_mat, w1p, b1p, w2p, b2p, w3p, b3p, w4p, b4p)


def _ln(x, w, b):
    mu = jnp.mean(x, axis=-1, keepdims=True)
    var = jnp.mean((x - mu) ** 2, axis=-1, keepdims=True)
    return (x - mu) / jnp.sqrt(var + 1e-5) * w + b


def _tr_body(h_ref, win_ref, wout_ref, w1_ref, w2_ref, inb_ref, outb_ref,
             l1b_ref, l2b_ref, ln1w_ref, ln1b_ref, ln2w_ref, ln2b_ref,
             ow_ref, ob_ref, out_ref, h_s, qkv_s, o_s, pj_s, ff_s):
    g = pl.program_id(0)
    s = g % NST
    l = g // NST

    @pl.when(g == 0)
    def _init():
        h_s[...] = h_ref[...]

    @pl.when(s < 6)
    def _qkv():
        part = lax.dot_general(h_s[...], win_ref[0],
                               (((1,), (1,)), ((), ())),
                               preferred_element_type=jnp.float32)
        qkv_s[pl.ds(s, 1)] = part[None]

    @pl.when(s == 6)
    def _attn():
        scale = jnp.float32(np.sqrt(DH).astype(np.float32))
        lrow = pl.ds(l, 1)
        q = jnp.concatenate([qkv_s[0], qkv_s[1]], axis=1) + inb_ref[lrow, :D]
        k = (jnp.concatenate([qkv_s[2], qkv_s[3]], axis=1)
             + inb_ref[lrow, D:2 * D])
        v = (jnp.concatenate([qkv_s[4], qkv_s[5]], axis=1)
             + inb_ref[lrow, 2 * D:3 * D])
        outs = []
        for hh in range(NHEAD):
            qh = q[:, hh * DH:(hh + 1) * DH]
            kh = k[:, hh * DH:(hh + 1) * DH]
            vh = v[:, hh * DH:(hh + 1) * DH]
            logits = lax.dot_general(qh, kh, (((1,), (1,)), ((), ())),
                                     preferred_element_type=jnp.float32)
            logits = logits / scale
            m = jnp.max(logits, axis=-1, keepdims=True)
            e = jnp.exp(logits - m)
            att = e / jnp.sum(e, axis=-1, keepdims=True)
            outs.append(lax.dot_general(att, vh, (((1,), (0,)), ((), ())),
                                        preferred_element_type=jnp.float32))
        o_s[...] = jnp.concatenate(outs, axis=1)

    @pl.when((s == 7) | (s == 8))
    def _proj():
        pj = lax.dot_general(o_s[...], wout_ref[0], (((1,), (1,)), ((), ())),
                             preferred_element_type=jnp.float32)
        pj_s[pl.ds(s - 7, 1)] = pj[None]

        @pl.when(s == 8)
        def _res1():
            lrow = pl.ds(l, 1)
            pj_full = (jnp.concatenate([pj_s[0], pj_s[1]], axis=1)
                       + outb_ref[lrow, :])
            h_s[...] = _ln(h_s[...] + pj_full, ln1w_ref[lrow, :],
                           ln1b_ref[lrow, :])

    @pl.when(s == 9)
    def _ff1():
        hid = lax.dot_general(h_s[...], w1_ref[0], (((1,), (1,)), ((), ())),
                              preferred_element_type=jnp.float32)
        ff_s[...] = jnp.maximum(hid + l1b_ref[pl.ds(l, 1), :], 0.0)

    @pl.when(s == 10)
    def _ff2():
        lrow = pl.ds(l, 1)
        ff2 = lax.dot_general(ff_s[...], w2_ref[0], (((1,), (1,)), ((), ())),
                              preferred_element_type=jnp.float32)
        ff2 = ff2 + l2b_ref[lrow, :]
        h2 = _ln(h_s[...] + ff2, ln2w_ref[lrow, :], ln2b_ref[lrow, :])
        h_s[...] = h2

        @pl.when(g == NST * NLAYERS - 1)
        def _final():
            res = lax.dot_general(h2, ow_ref[...], (((1,), (1,)), ((), ())),
                                  preferred_element_type=jnp.float32)
            out_ref[...] = res + ob_ref[...]


def _transformer_call(h, win_r, wout_r, w1, w2, inb, outb, l1b, l2b,
                      ln1w, ln1b, ln2w, ln2b, ow, ob2):
    nsteps = NST * NLAYERS
    const2 = lambda g: (0, 0)
    win_idx = lambda g: (6 * (g // NST) + jnp.minimum(g % NST, 5), 0, 0)
    wout_idx = lambda g: (jnp.where(
        g % NST >= 8, 2 * (g // NST) + 1,
        jnp.where(g % NST >= 7, 2 * (g // NST),
                  jnp.maximum(2 * (g // NST) - 1, 0))), 0, 0)
    w1_idx = lambda g: (jnp.where(g % NST >= 9, g // NST,
                                  jnp.maximum(g // NST - 1, 0)), 0, 0)
    w2_idx = lambda g: (jnp.where(g % NST >= 10, g // NST,
                                  jnp.maximum(g // NST - 1, 0)), 0, 0)
    in_specs = [
        pl.BlockSpec((B, D), const2),
        pl.BlockSpec((1, HC, D), win_idx),
        pl.BlockSpec((1, HC, D), wout_idx),
        pl.BlockSpec((1, DFF, D), w1_idx),
        pl.BlockSpec((1, D, DFF), w2_idx),
        pl.BlockSpec((NLAYERS, 3 * D), const2),
        pl.BlockSpec((NLAYERS, D), const2),
        pl.BlockSpec((NLAYERS, DFF), const2),
        pl.BlockSpec((NLAYERS, D), const2),
        pl.BlockSpec((NLAYERS, D), const2),
        pl.BlockSpec((NLAYERS, D), const2),
        pl.BlockSpec((NLAYERS, D), const2),
        pl.BlockSpec((NLAYERS, D), const2),
        pl.BlockSpec((LBL, D), const2),
        pl.BlockSpec((1, LBL), const2),
    ]
    return pl.pallas_call(
        _tr_body,
        grid=(nsteps,),
        in_specs=in_specs,
        out_specs=pl.BlockSpec((B, LBL), const2),
        out_shape=jax.ShapeDtypeStruct((B, LBL), jnp.float32),
        scratch_shapes=[
            pltpu.VMEM((B, D), jnp.float32),
            pltpu.VMEM((6, B, HC), jnp.float32),
            pltpu.VMEM((B, D), jnp.float32),
            pltpu.VMEM((2, B, HC), jnp.float32),
            pltpu.VMEM((B, DFF), jnp.float32),
        ],
        compiler_params=pltpu.CompilerParams(
            dimension_semantics=("arbitrary",),
        ),
    )(h, win_r, wout_r, w1, w2, inb, outb, l1b, l2b, ln1w, ln1b, ln2w, ln2b,
      ow, ob2)


def kernel(inputs, edge_index, edge_attr, gcn_w1, gcn_b1, gcn_w2, gcn_b2,
           gcn_w3, gcn_b3, gcn_w4, gcn_b4, tr_in_w, tr_in_b, tr_out_w,
           tr_out_b, tr_l1_w, tr_l1_b, tr_l2_w, tr_l2_b, tr_ln1_w, tr_ln1_b,
           tr_ln2_w, tr_ln2_b, out_w, out_b):
    x3 = jnp.transpose(inputs, (2, 0, 1))
    x3 = jnp.pad(x3, ((0, NP - ENC), (0, 0), (0, FP - SEQ)))
    a_mat = _a_build_sc(edge_index, edge_attr)

    z4 = _graph_gcn_call(x3, a_mat,
                         gcn_w1, gcn_b1.reshape(1, -1),
                         gcn_w2, gcn_b2.reshape(1, -1),
                         gcn_w3, gcn_b3.reshape(1, -1),
                         gcn_w4, gcn_b4.reshape(1, -1))
    h = z4[:ENC, :, :SEQ // 6].transpose(1, 0, 2).reshape(B, D)

    win_r = tr_in_w.reshape(NLAYERS * 6, HC, D)
    wout_r = tr_out_w.reshape(NLAYERS * 2, HC, D)
    out = _transformer_call(
        h, win_r, wout_r, tr_l1_w, tr_l2_w, tr_in_b, tr_out_b, tr_l1_b,
        tr_l2_b, tr_ln1_w, tr_ln1_b, tr_ln2_w, tr_ln2_b, out_w,
        out_b.reshape(1, LBL))
    return out.reshape(B, 1, LBL)
